# baseline (speedup 1.0000x reference)
import functools

import jax
import jax.numpy as jnp
import numpy as np
from jax import lax
from jax.experimental import pallas as pl
from jax.experimental.pallas import tpu as pltpu
from jax.experimental.pallas import tpu_sc as plsc

DIM = 96
HH = 128
WW = 128
NH = 6
NP = 4
B = 2
HEAD_D = DIM // NH
NQ = HH * WW
ROWS = B * NQ
HROWS = NQ
NTAP = HROWS * DIM
TAP_ROW = 128
SC_ROWS = NTAP // TAP_ROW
NC, NS = 2, 16
NW = NC * NS
RPW = SC_ROWS // NW


def _build_consts():
    Px = np.zeros((48, 96), np.float32)
    Py = np.zeros((48, 96), np.float32)
    Pa = np.zeros((24, 96), np.float32)
    for h in range(NH):
        for p in range(NP):
            for k in range(4):
                c = h * 16 + p * 4 + k
                Px[h * 8 + p * 2 + 0, c] = 1
                Py[h * 8 + p * 2 + 1, c] = 1
                Pa[h * 4 + p, c] = 1
    G96 = np.zeros((96, 96), np.float32)
    for i in range(96):
        for j in range(96):
            if i // 16 == j // 16:
                G96[i, j] = 0.25
    cols = np.arange(96)
    kx = (cols % 4 % 2).astype(np.float32)[None, :]
    ky = (cols % 4 // 2).astype(np.float32)[None, :]
    hcol = (cols // 16).astype(np.int32)[None, :]
    h = cols // 16
    p = cols % 16 // 4
    cmx = h * 8 + p * 2
    cmy = h * 8 + p * 2 + 1
    cma = h * 4 + p
    return Px, Py, Pa, G96, kx, ky, hcol, cmx, cmy, cma


(_PX, _PY, _PA, _G96, _KX, _KY, _HCOL, _CMX, _CMY, _CMA) = _build_consts()

R1 = 1024


def _stage1_body(q_ref, v_ref, w3_ref, b3_ref, g_ref, wv_ref, bv_ref,
                 kx_ref, ky_ref, hc_ref, idx_ref, wgt_ref, vp_ref):
    q = q_ref[...]
    x3 = jnp.dot(q, w3_ref[...], preferred_element_type=jnp.float32) + b3_ref[...]
    px = x3[:, 0:96] * WW - 0.5
    py = x3[:, 96:192] * HH - 0.5
    lg = x3[:, 192:288]
    m = jnp.max(lg, axis=1, keepdims=True)
    e = jnp.exp(lg - m)
    s = jnp.dot(e, g_ref[...], preferred_element_type=jnp.float32,
                precision=jax.lax.Precision.HIGHEST)
    a96 = e / s
    x0 = jnp.floor(px)
    y0 = jnp.floor(py)
    fx = px - x0
    fy = py - y0
    kx = kx_ref[...]
    ky = ky_ref[...]
    xi = x0 + kx
    yi = y0 + ky
    wx = jnp.where(kx > 0.5, fx, 1.0 - fx)
    wy = jnp.where(ky > 0.5, fy, 1.0 - fy)
    valid = ((xi >= 0) & (xi < 128) & (yi >= 0) & (yi < 128)).astype(jnp.float32)
    wgt_ref[...] = wx * wy * a96 * valid
    xc = jnp.clip(xi, 0, 127).astype(jnp.int32)
    yc = jnp.clip(yi, 0, 127).astype(jnp.int32)
    idx_ref[...] = (yc * 128 + xc) * NH + hc_ref[...]
    vp_ref[...] = jnp.dot(v_ref[...], wv_ref[...],
                          preferred_element_type=jnp.float32) + bv_ref[...]


def _stage3_body(x_ref, wo_ref, bo_ref, o_ref):
    o_ref[...] = jnp.dot(x_ref[...], wo_ref[...],
                         preferred_element_type=jnp.float32) + bo_ref[...]


LCH = 64
NCHUNK = RPW // LCH
GR = 8
NGRP = LCH // GR
QH_G = GR * TAP_ROW // 16


def _make_sc_gather():
    mesh = plsc.VectorSubcoreMesh(core_axis_name="c", subcore_axis_name="s")

    @functools.partial(
        pl.kernel,
        out_type=jax.ShapeDtypeStruct((HROWS * NH, HEAD_D), jnp.float32),
        mesh=mesh,
        compiler_params=pltpu.CompilerParams(use_tc_tiling_on_sc=False,
                                             needs_layout_passes=False),
        scratch_types=[
            pltpu.VMEM((LCH, TAP_ROW), jnp.int32),
            pltpu.VMEM((LCH * TAP_ROW,), jnp.float32),
            pltpu.VMEM((GR * TAP_ROW, HEAD_D), jnp.float32),
            pltpu.VMEM((GR * TAP_ROW, HEAD_D), jnp.float32),
            pltpu.VMEM((LCH * TAP_ROW // 16, HEAD_D), jnp.float32),
            pltpu.SemaphoreType.DMA,
        ],
    )
    def sc_gather(tab_hbm, idx_hbm, wgt_hbm, out_hbm,
                  idx_v, wgt_v, buf_a, buf_b, out_v, sem):
        wid = lax.axis_index("s") * NC + lax.axis_index("c")
        lanes = lax.iota(jnp.int32, 16)
        lanes16 = lanes * 16
        cfull = [jnp.full((16,), d, jnp.int32) for d in range(HEAD_D)]

        def fire_group(g, buf):
            for lr in range(GR):
                pltpu.async_copy(
                    tab_hbm.at[idx_v.at[g * GR + lr]],
                    buf.at[pl.ds(lr * TAP_ROW, TAP_ROW)],
                    sem,
                )

        def wait_group(buf):
            pltpu.make_async_copy(
                tab_hbm.at[pl.ds(0, GR * TAP_ROW)], buf, sem).wait()

        def compute_group(g, buf):
            def one_qh(j):
                wv = wgt_v[pl.ds((g * QH_G + j) * 16, 16)]
                prods = [buf[j * 16 + t, :] * wv[t] for t in range(16)]
                while len(prods) > 1:
                    prods = [prods[i] + prods[i + 1]
                             for i in range(0, len(prods), 2)]
                out_v[g * QH_G + j, :] = prods[0]

            def blk(b2, carry):
                one_qh(b2 * 2)
                one_qh(b2 * 2 + 1)
                return carry

            lax.fori_loop(0, QH_G // 2, blk, 0)

        def chunk(c, carry):
            base = wid * RPW + c * LCH
            pltpu.sync_copy(idx_hbm.at[pl.ds(base, LCH)], idx_v)
            pltpu.sync_copy(wgt_hbm.at[pl.ds(base * TAP_ROW, LCH * TAP_ROW)],
                            wgt_v)
            fire_group(0, buf_a)

            def pair(p, carry2):
                g0 = 2 * p
                fire_group(g0 + 1, buf_b)
                wait_group(buf_a)
                compute_group(g0, buf_a)

                @pl.when(g0 + 2 < NGRP)
                def _():
                    fire_group(g0 + 2, buf_a)

                wait_group(buf_b)
                compute_group(g0 + 1, buf_b)
                return carry2

            lax.fori_loop(0, NGRP // 2, pair, 0)
            pltpu.sync_copy(out_v,
                            out_hbm.at[pl.ds(base * (TAP_ROW // 16),
                                             LCH * TAP_ROW // 16)])
            return carry

        lax.fori_loop(0, NCHUNK, chunk, 0)

    return sc_gather


def _half_pipeline(q, v, W3, b3, G96c, Wv, bv2, Wo, bo2, kxc, kyc, hcc, sc_fn):
    grid1 = HROWS // R1
    idx, wgt, vproj = pl.pallas_call(
        _stage1_body,
        grid=(grid1,),
        in_specs=[
            pl.BlockSpec((R1, DIM), lambda i: (i, 0)),
            pl.BlockSpec((R1, DIM), lambda i: (i, 0)),
            pl.BlockSpec((DIM, 288), lambda i: (0, 0)),
            pl.BlockSpec((1, 288), lambda i: (0, 0)),
            pl.BlockSpec((96, 96), lambda i: (0, 0)),
            pl.BlockSpec((DIM, DIM), lambda i: (0, 0)),
            pl.BlockSpec((1, DIM), lambda i: (0, 0)),
            pl.BlockSpec((1, 96), lambda i: (0, 0)),
            pl.BlockSpec((1, 96), lambda i: (0, 0)),
            pl.BlockSpec((1, 96), lambda i: (0, 0)),
        ],
        out_specs=[
            pl.BlockSpec((R1, 96), lambda i: (i, 0)),
            pl.BlockSpec((R1, 96), lambda i: (i, 0)),
            pl.BlockSpec((R1, DIM), lambda i: (i, 0)),
        ],
        out_shape=[
            jax.ShapeDtypeStruct((HROWS, 96), jnp.int32),
            jax.ShapeDtypeStruct((HROWS, 96), jnp.float32),
            jax.ShapeDtypeStruct((HROWS, DIM), jnp.float32),
        ],
    )(q, v, W3, b3, G96c, Wv, bv2, kxc, kyc, hcc)

    tab = vproj.reshape(HROWS * NH, HEAD_D)
    idx_rows = idx.reshape(SC_ROWS, TAP_ROW)
    out16 = sc_fn(tab, idx_rows, wgt.reshape(-1))
    out96 = out16.reshape(HROWS, DIM)

    return pl.pallas_call(
        _stage3_body,
        grid=(grid1,),
        in_specs=[
            pl.BlockSpec((R1, DIM), lambda i: (i, 0)),
            pl.BlockSpec((DIM, DIM), lambda i: (0, 0)),
            pl.BlockSpec((1, DIM), lambda i: (0, 0)),
        ],
        out_specs=pl.BlockSpec((R1, DIM), lambda i: (i, 0)),
        out_shape=jax.ShapeDtypeStruct((HROWS, DIM), jnp.float32),
    )(out96, Wo, bo2)


def kernel(query, value, Ws, bs_, Wa, ba, Wv, bv, Wo, bo):
    q = jnp.transpose(query, (0, 2, 3, 1)).reshape(ROWS, DIM)
    v = jnp.transpose(value, (0, 2, 3, 1)).reshape(ROWS, DIM)

    W3 = jnp.concatenate([Ws[:, _CMX], Ws[:, _CMY], Wa[:, _CMA]], axis=1)
    b3 = jnp.concatenate([bs_[_CMX], bs_[_CMY], ba[_CMA]])[None, :]

    G96c = jnp.asarray(_G96)
    kxc, kyc, hcc = jnp.asarray(_KX), jnp.asarray(_KY), jnp.asarray(_HCOL)
    bv2, bo2 = bv[None, :], bo[None, :]
    sc_fn = _make_sc_gather()
    outs = [
        _half_pipeline(q[h * HROWS:(h + 1) * HROWS],
                       v[h * HROWS:(h + 1) * HROWS],
                       W3, b3, G96c, Wv, bv2, Wo, bo2, kxc, kyc, hcc, sc_fn)
        for h in range(B)
    ]
    return jnp.stack(outs, axis=0)


# speedup vs baseline: 1.0936x; 1.0936x over previous
import functools

import jax
import jax.numpy as jnp
import numpy as np
from jax import lax
from jax.experimental import pallas as pl
from jax.experimental.pallas import tpu as pltpu
from jax.experimental.pallas import tpu_sc as plsc

DIM = 96
HH = 128
WW = 128
NH = 6
NP = 4
B = 2
HEAD_D = DIM // NH
NQ = HH * WW
ROWS = B * NQ
HROWS = ROWS
NTAP = HROWS * DIM
TAP_ROW = 128
SC_ROWS = NTAP // TAP_ROW
NC, NS = 2, 16
NW = NC * NS
RPW = SC_ROWS // NW


def _build_consts():
    Px = np.zeros((48, 96), np.float32)
    Py = np.zeros((48, 96), np.float32)
    Pa = np.zeros((24, 96), np.float32)
    for h in range(NH):
        for p in range(NP):
            for k in range(4):
                c = h * 16 + p * 4 + k
                Px[h * 8 + p * 2 + 0, c] = 1
                Py[h * 8 + p * 2 + 1, c] = 1
                Pa[h * 4 + p, c] = 1
    G96 = np.zeros((96, 96), np.float32)
    for i in range(96):
        for j in range(96):
            if i // 16 == j // 16:
                G96[i, j] = 0.25
    cols = np.arange(96)
    kx = (cols % 4 % 2).astype(np.float32)[None, :]
    ky = (cols % 4 // 2).astype(np.float32)[None, :]
    hcol = (cols // 16).astype(np.int32)[None, :]
    h = cols // 16
    p = cols % 16 // 4
    cmx = h * 8 + p * 2
    cmy = h * 8 + p * 2 + 1
    cma = h * 4 + p
    return Px, Py, Pa, G96, kx, ky, hcol, cmx, cmy, cma


(_PX, _PY, _PA, _G96, _KX, _KY, _HCOL, _CMX, _CMY, _CMA) = _build_consts()

R1 = 1024


def _stage1_body(q_ref, v_ref, w3_ref, b3_ref, g_ref, wv_ref, bv_ref,
                 kx_ref, ky_ref, hc_ref, idx_ref, wgt_ref, vp_ref):
    q = q_ref[...]
    x3 = jnp.dot(q, w3_ref[...], preferred_element_type=jnp.float32) + b3_ref[...]
    px = x3[:, 0:96] * WW - 0.5
    py = x3[:, 96:192] * HH - 0.5
    lg = x3[:, 192:288]
    m = jnp.max(lg, axis=1, keepdims=True)
    e = jnp.exp(lg - m)
    s = jnp.dot(e, g_ref[...], preferred_element_type=jnp.float32,
                precision=jax.lax.Precision.HIGHEST)
    a96 = e / s
    x0 = jnp.floor(px)
    y0 = jnp.floor(py)
    fx = px - x0
    fy = py - y0
    kx = kx_ref[...]
    ky = ky_ref[...]
    xi = x0 + kx
    yi = y0 + ky
    wx = jnp.where(kx > 0.5, fx, 1.0 - fx)
    wy = jnp.where(ky > 0.5, fy, 1.0 - fy)
    valid = ((xi >= 0) & (xi < 128) & (yi >= 0) & (yi < 128)).astype(jnp.float32)
    wgt_ref[...] = wx * wy * a96 * valid
    xc = jnp.clip(xi, 0, 127).astype(jnp.int32)
    yc = jnp.clip(yi, 0, 127).astype(jnp.int32)
    b = pl.program_id(0) // (NQ // R1)
    idx_ref[...] = (b * (NQ * NH) + (yc * 128 + xc) * NH) + hc_ref[...]
    vp_ref[...] = jnp.dot(v_ref[...], wv_ref[...],
                          preferred_element_type=jnp.float32) + bv_ref[...]


def _stage3_body(x_ref, wo_ref, bo_ref, o_ref):
    o_ref[...] = jnp.dot(x_ref[...], wo_ref[...],
                         preferred_element_type=jnp.float32) + bo_ref[...]


LCH = 64
NCHUNK = RPW // LCH
GR = 8
NGRP = LCH // GR
QH_G = GR * TAP_ROW // 16


def _make_sc_gather():
    mesh = plsc.VectorSubcoreMesh(core_axis_name="c", subcore_axis_name="s")

    @functools.partial(
        pl.kernel,
        out_type=jax.ShapeDtypeStruct((HROWS * NH, HEAD_D), jnp.float32),
        mesh=mesh,
        compiler_params=pltpu.CompilerParams(use_tc_tiling_on_sc=False,
                                             needs_layout_passes=False),
        scratch_types=[
            pltpu.VMEM((LCH, TAP_ROW), jnp.int32),
            pltpu.VMEM((LCH * TAP_ROW,), jnp.float32),
            pltpu.VMEM((GR * TAP_ROW, HEAD_D), jnp.float32),
            pltpu.VMEM((GR * TAP_ROW, HEAD_D), jnp.float32),
            pltpu.VMEM((LCH * TAP_ROW // 16, HEAD_D), jnp.float32),
            pltpu.SemaphoreType.DMA,
        ],
    )
    def sc_gather(tab_hbm, idx_hbm, wgt_hbm, out_hbm,
                  idx_v, wgt_v, buf_a, buf_b, out_v, sem):
        wid = lax.axis_index("s") * NC + lax.axis_index("c")
        lanes = lax.iota(jnp.int32, 16)
        lanes16 = lanes * 16
        cfull = [jnp.full((16,), d, jnp.int32) for d in range(HEAD_D)]

        def fire_group(g, buf):
            for lr in range(GR):
                pltpu.async_copy(
                    tab_hbm.at[idx_v.at[g * GR + lr]],
                    buf.at[pl.ds(lr * TAP_ROW, TAP_ROW)],
                    sem,
                )

        def wait_group(buf):
            pltpu.make_async_copy(
                tab_hbm.at[pl.ds(0, GR * TAP_ROW)], buf, sem).wait()

        def compute_group(g, buf):
            def one_qh(j):
                wv = wgt_v[pl.ds((g * QH_G + j) * 16, 16)]
                prods = [buf[j * 16 + t, :] * wv[t] for t in range(16)]
                while len(prods) > 1:
                    prods = [prods[i] + prods[i + 1]
                             for i in range(0, len(prods), 2)]
                out_v[g * QH_G + j, :] = prods[0]

            def blk(b2, carry):
                one_qh(b2 * 2)
                one_qh(b2 * 2 + 1)
                return carry

            lax.fori_loop(0, QH_G // 2, blk, 0)

        def chunk(c, carry):
            base = wid * RPW + c * LCH
            pltpu.sync_copy(idx_hbm.at[pl.ds(base, LCH)], idx_v)
            pltpu.sync_copy(wgt_hbm.at[pl.ds(base * TAP_ROW, LCH * TAP_ROW)],
                            wgt_v)
            fire_group(0, buf_a)

            def pair(p, carry2):
                g0 = 2 * p
                fire_group(g0 + 1, buf_b)
                wait_group(buf_a)
                compute_group(g0, buf_a)

                @pl.when(g0 + 2 < NGRP)
                def _():
                    fire_group(g0 + 2, buf_a)

                wait_group(buf_b)
                compute_group(g0 + 1, buf_b)
                return carry2

            lax.fori_loop(0, NGRP // 2, pair, 0)
            pltpu.sync_copy(out_v,
                            out_hbm.at[pl.ds(base * (TAP_ROW // 16),
                                             LCH * TAP_ROW // 16)])
            return carry

        lax.fori_loop(0, NCHUNK, chunk, 0)

    return sc_gather


def _half_pipeline(q, v, W3, b3, G96c, Wv, bv2, Wo, bo2, kxc, kyc, hcc, sc_fn):
    grid1 = HROWS // R1
    idx, wgt, vproj = pl.pallas_call(
        _stage1_body,
        grid=(grid1,),
        in_specs=[
            pl.BlockSpec((R1, DIM), lambda i: (i, 0)),
            pl.BlockSpec((R1, DIM), lambda i: (i, 0)),
            pl.BlockSpec((DIM, 288), lambda i: (0, 0)),
            pl.BlockSpec((1, 288), lambda i: (0, 0)),
            pl.BlockSpec((96, 96), lambda i: (0, 0)),
            pl.BlockSpec((DIM, DIM), lambda i: (0, 0)),
            pl.BlockSpec((1, DIM), lambda i: (0, 0)),
            pl.BlockSpec((1, 96), lambda i: (0, 0)),
            pl.BlockSpec((1, 96), lambda i: (0, 0)),
            pl.BlockSpec((1, 96), lambda i: (0, 0)),
        ],
        out_specs=[
            pl.BlockSpec((R1, 96), lambda i: (i, 0)),
            pl.BlockSpec((R1, 96), lambda i: (i, 0)),
            pl.BlockSpec((R1, DIM), lambda i: (i, 0)),
        ],
        out_shape=[
            jax.ShapeDtypeStruct((HROWS, 96), jnp.int32),
            jax.ShapeDtypeStruct((HROWS, 96), jnp.float32),
            jax.ShapeDtypeStruct((HROWS, DIM), jnp.float32),
        ],
    )(q, v, W3, b3, G96c, Wv, bv2, kxc, kyc, hcc)

    tab = vproj.reshape(HROWS * NH, HEAD_D)
    idx_rows = idx.reshape(SC_ROWS, TAP_ROW)
    out16 = sc_fn(tab, idx_rows, wgt.reshape(-1))
    out96 = out16.reshape(HROWS, DIM)

    return pl.pallas_call(
        _stage3_body,
        grid=(grid1,),
        in_specs=[
            pl.BlockSpec((R1, DIM), lambda i: (i, 0)),
            pl.BlockSpec((DIM, DIM), lambda i: (0, 0)),
            pl.BlockSpec((1, DIM), lambda i: (0, 0)),
        ],
        out_specs=pl.BlockSpec((R1, DIM), lambda i: (i, 0)),
        out_shape=jax.ShapeDtypeStruct((HROWS, DIM), jnp.float32),
    )(out96, Wo, bo2)


def kernel(query, value, Ws, bs_, Wa, ba, Wv, bv, Wo, bo):
    q = jnp.transpose(query, (0, 2, 3, 1)).reshape(ROWS, DIM)
    v = jnp.transpose(value, (0, 2, 3, 1)).reshape(ROWS, DIM)

    W3 = jnp.concatenate([Ws[:, _CMX], Ws[:, _CMY], Wa[:, _CMA]], axis=1)
    b3 = jnp.concatenate([bs_[_CMX], bs_[_CMY], ba[_CMA]])[None, :]

    G96c = jnp.asarray(_G96)
    kxc, kyc, hcc = jnp.asarray(_KX), jnp.asarray(_KY), jnp.asarray(_HCOL)
    bv2, bo2 = bv[None, :], bo[None, :]
    sc_fn = _make_sc_gather()
    out = _half_pipeline(q, v, W3, b3, G96c, Wv, bv2, Wo, bo2,
                         kxc, kyc, hcc, sc_fn)
    return out.reshape(B, NQ, DIM)
